# baseline (speedup 1.0000x reference)
import functools

import jax
import jax.numpy as jnp
from jax import lax
from jax.experimental import pallas as pl
from jax.experimental.pallas import tpu as pltpu
from jax.experimental.pallas import tpu_sc as plsc

N = 10000
E = 160000
D_IN = 256
D_HID = 512
EPS = 1e-5

NPAD = 10240
EPAD = 163840
N_WORKERS = 32
EDGES_PER_W = EPAD // N_WORKERS
BLK_E = 64
N_BLKS = EDGES_PER_W // BLK_E
ROWS_PER_TILE = NPAD // 16
R = 1024


NBUF = 4
N_GRP = N_BLKS // NBUF


@functools.lru_cache(maxsize=None)
def _make_sc_segsum(C, gather, d_table=None, col0=0):
    mesh = plsc.VectorSubcoreMesh(core_axis_name="c", subcore_axis_name="s")

    scratch = [
        pltpu.VMEM((N_BLKS, BLK_E), jnp.int32),
        pltpu.VMEM((NBUF, BLK_E, C), jnp.float32),
        pltpu.VMEM_SHARED((NPAD, C), jnp.float32),
        pltpu.SemaphoreType.DMA,
    ] + [pltpu.SemaphoreType.DMA] * NBUF \
      + [pltpu.SemaphoreType.DMA] * NBUF
    if gather:
        scratch.insert(0, pltpu.VMEM((EDGES_PER_W,), jnp.int32))

    def body(*refs):
        if gather:
            (table, src_hbm, dst_hbm, zeros_hbm, out_hbm,
             srcb, dstb, rows, acc, zsem, *sems) = refs
        else:
            (table, dst_hbm, zeros_hbm, out_hbm,
             dstb, rows, acc, zsem, *sems) = refs
        gsem = sems[:NBUF]
        ssem = sems[NBUF:]

        cid = lax.axis_index("c")
        sid = lax.axis_index("s")
        wid = sid * 2 + cid
        r0 = pl.multiple_of(sid * ROWS_PER_TILE, 8)
        i0 = pl.multiple_of(wid * N_BLKS, 8)

        zdesc = pltpu.make_async_copy(zeros_hbm.at[pl.ds(r0, ROWS_PER_TILE)],
                                      acc.at[pl.ds(r0, ROWS_PER_TILE)], zsem)
        zdesc.start()
        pltpu.sync_copy(dst_hbm.at[pl.ds(i0, N_BLKS)], dstb)
        if gather:
            e0 = pl.multiple_of(wid * EDGES_PER_W, 8)
            pltpu.sync_copy(src_hbm.at[pl.ds(e0, EDGES_PER_W)], srcb)
        else:
            for b in range(NBUF):
                pltpu.sync_copy(table, rows.at[b])
        zdesc.wait()
        plsc.subcore_barrier()

        def g_desc(i, b):
            off = pl.multiple_of(i * BLK_E, 8)
            return pltpu.make_async_copy(
                table.at[srcb.at[pl.ds(off, BLK_E)], pl.ds(col0, C)],
                rows.at[b], gsem[b])

        def s_desc(i, b):
            return pltpu.make_async_copy(rows.at[b], acc.at[dstb.at[i]],
                                         ssem[b])

        if gather:
            for b in range(NBUF):
                g_desc(b, b).start()

            def group(g, carry):
                ib = g * NBUF
                for b in range(NBUF):
                    g_desc(ib + b, b).wait()
                    s_desc(ib + b, b).start(add=True)

                @pl.when(g < N_GRP - 1)
                def _():
                    for b in range(NBUF):
                        s_desc(ib + b, b).wait()
                        g_desc(ib + NBUF + b, b).start()
                return carry

            lax.fori_loop(0, N_GRP, group, 0)
            for b in range(NBUF):
                s_desc((N_GRP - 1) * NBUF + b, b).wait()
        else:
            def group(g, carry):
                ib = g * NBUF
                for b in range(NBUF):
                    s_desc(ib + b, b).start(add=True)
                for b in range(NBUF):
                    s_desc(ib + b, b).wait()
                return carry

            lax.fori_loop(0, N_GRP, group, 0)

        plsc.subcore_barrier()

        o0 = pl.multiple_of(cid * NPAD + r0, 8)
        pltpu.sync_copy(acc.at[pl.ds(r0, ROWS_PER_TILE)],
                        out_hbm.at[pl.ds(o0, ROWS_PER_TILE)])

    return pl.kernel(
        body,
        out_type=jax.ShapeDtypeStruct((2 * NPAD, C), jnp.float32),
        mesh=mesh,
        scratch_types=scratch,
    )


def _make_sc_hist(gather):
    mesh = plsc.VectorSubcoreMesh(core_axis_name="c", subcore_axis_name="s")

    scratch = [
        pltpu.VMEM((N_BLKS, BLK_E), jnp.int32),
        pltpu.VMEM((NPAD,), jnp.float32),
    ]
    if gather:
        scratch.insert(0, pltpu.VMEM((EDGES_PER_W,), jnp.int32))
        scratch.append(pltpu.VMEM((NPAD // 128, 128), jnp.float32))

    def body(*refs):
        if gather:
            (ptab_hbm, src_hbm, dst_hbm, zeros_hbm, out_hbm,
             srcb, dstb, acc, ptab) = refs
        else:
            (dst_hbm, zeros_hbm, out_hbm, dstb, acc) = refs

        cid = lax.axis_index("c")
        sid = lax.axis_index("s")
        wid = sid * 2 + cid
        i0 = pl.multiple_of(wid * N_BLKS, 8)

        pltpu.sync_copy(zeros_hbm, acc)
        pltpu.sync_copy(dst_hbm.at[pl.ds(i0, N_BLKS)], dstb)
        if gather:
            e0 = pl.multiple_of(wid * EDGES_PER_W, 8)
            pltpu.sync_copy(src_hbm.at[pl.ds(e0, EDGES_PER_W)], srcb)
            pltpu.sync_copy(ptab_hbm, ptab)
        ones = jnp.full((16,), 1.0, jnp.float32)

        def step(j, carry):
            for l in range(BLK_E // 16):
                dv = dstb[j, pl.ds(l * 16, 16)]
                if gather:
                    sv = srcb[pl.ds(j * BLK_E + l * 16, 16)]
                    pv = plsc.load_gather(
                        ptab, [lax.shift_right_logical(sv, 7),
                               lax.bitwise_and(sv, 127)])
                else:
                    pv = ones
                plsc.addupdate_scatter(acc, [dv], pv)
            return carry

        lax.fori_loop(0, N_BLKS, step, 0)
        pltpu.sync_copy(acc, out_hbm.at[wid])

    return pl.kernel(
        body,
        out_type=jax.ShapeDtypeStruct((N_WORKERS, NPAD), jnp.float32),
        mesh=mesh,
        scratch_types=scratch,
        compiler_params=pltpu.CompilerParams(needs_layout_passes=False),
    )


_sc_degree = _make_sc_hist(gather=False)
_sc_psum = _make_sc_hist(gather=True)


def _mm_self(xin, W, b2d):
    d_in = xin.shape[1]

    def body(x_ref, w_ref, b_ref, o_ref):
        o_ref[...] = jnp.dot(x_ref[...], w_ref[...],
                             preferred_element_type=jnp.float32) + b_ref[...]

    return pl.pallas_call(
        body,
        grid=(NPAD // R,),
        in_specs=[
            pl.BlockSpec((R, d_in), lambda i: (i, 0)),
            pl.BlockSpec((d_in, D_HID), lambda i: (0, 0)),
            pl.BlockSpec((1, D_HID), lambda i: (0, 0)),
        ],
        out_specs=pl.BlockSpec((R, D_HID), lambda i: (i, 0)),
        out_shape=jax.ShapeDtypeStruct((N, D_HID), jnp.float32),
    )(xin, W, b2d)


def _mm_acc(h, part, degp, Wc, want_stats):
    def body(h_ref, part_ref, degp_ref, w_ref, *outs):
        aggc = part_ref[0] + part_ref[1]
        deg = jnp.sum(degp_ref[...], axis=0)[:, None]
        inv = 1.0 / jnp.maximum(deg, 1.0)
        hn = h_ref[...] + jnp.dot(aggc * inv, w_ref[...],
                                  preferred_element_type=jnp.float32)
        outs[0][...] = hn
        if want_stats:
            stats_ref = outs[1]
            i = pl.program_id(0)

            @pl.when(i == 0)
            def _():
                stats_ref[...] = jnp.zeros_like(stats_ref)

            row = i * R + jax.lax.broadcasted_iota(jnp.int32, (R, 1), 0)
            hm = jnp.where(row < N, hn, 0.0)
            s1 = jnp.broadcast_to(jnp.sum(hm, axis=0, keepdims=True),
                                  (8, D_HID))
            s2 = jnp.broadcast_to(jnp.sum(hm * hm, axis=0, keepdims=True),
                                  (8, D_HID))
            stats_ref[...] += jnp.concatenate([s1, s2], axis=0)

    out_specs = [pl.BlockSpec((R, D_HID), lambda i: (i, 0))]
    out_shape = [jax.ShapeDtypeStruct((N, D_HID), jnp.float32)]
    if want_stats:
        out_specs.append(pl.BlockSpec((16, D_HID), lambda i: (0, 0)))
        out_shape.append(jax.ShapeDtypeStruct((16, D_HID), jnp.float32))
    return pl.pallas_call(
        body,
        grid=(NPAD // R,),
        in_specs=[
            pl.BlockSpec((R, D_HID), lambda i: (i, 0)),
            pl.BlockSpec((2, R, 128), lambda i: (0, i, 0)),
            pl.BlockSpec((N_WORKERS, R), lambda i: (0, i)),
            pl.BlockSpec((128, D_HID), lambda i: (0, 0)),
        ],
        out_specs=out_specs,
        out_shape=out_shape,
        input_output_aliases={0: 0},
    )(h, part, degp, Wc)


def _bn_relu(h, stats, g2d, beta2d):
    def body(h_ref, stats_ref, g_ref, beta_ref, o_ref):
        mu = stats_ref[0:1] * (1.0 / N)
        msq = stats_ref[8:9] * (1.0 / N)
        var = msq - mu * mu
        rstd = 1.0 / jnp.sqrt(var + EPS)
        o_ref[...] = jnp.maximum(
            (h_ref[...] - mu) * (rstd * g_ref[...]) + beta_ref[...], 0.0)

    return pl.pallas_call(
        body,
        grid=(NPAD // R,),
        in_specs=[
            pl.BlockSpec((R, D_HID), lambda i: (i, 0)),
            pl.BlockSpec((16, D_HID), lambda i: (0, 0)),
            pl.BlockSpec((1, D_HID), lambda i: (0, 0)),
            pl.BlockSpec((1, D_HID), lambda i: (0, 0)),
        ],
        out_specs=pl.BlockSpec((R, D_HID), lambda i: (i, 0)),
        out_shape=jax.ShapeDtypeStruct((N, D_HID), jnp.float32),
    )(h, stats, g2d, beta2d)


def _bn_relu_proj(h, stats, g2d, beta2d, wcat, bcat):
    def body(h_ref, stats_ref, g_ref, beta_ref, w_ref, b_ref, o_ref, sp_ref):
        mu = stats_ref[0:1] * (1.0 / N)
        msq = stats_ref[8:9] * (1.0 / N)
        var = msq - mu * mu
        rstd = 1.0 / jnp.sqrt(var + EPS)
        hn = jnp.maximum(
            (h_ref[...] - mu) * (rstd * g_ref[...]) + beta_ref[...], 0.0)
        o_ref[...] = hn
        sp_ref[...] = jnp.dot(hn, w_ref[...],
                              preferred_element_type=jnp.float32) + b_ref[...]

    return pl.pallas_call(
        body,
        grid=(NPAD // R,),
        in_specs=[
            pl.BlockSpec((R, D_HID), lambda i: (i, 0)),
            pl.BlockSpec((16, D_HID), lambda i: (0, 0)),
            pl.BlockSpec((1, D_HID), lambda i: (0, 0)),
            pl.BlockSpec((1, D_HID), lambda i: (0, 0)),
            pl.BlockSpec((D_HID, 128), lambda i: (0, 0)),
            pl.BlockSpec((1, 128), lambda i: (0, 0)),
        ],
        out_specs=[
            pl.BlockSpec((R, D_HID), lambda i: (i, 0)),
            pl.BlockSpec((R, 128), lambda i: (i, 0)),
        ],
        out_shape=[
            jax.ShapeDtypeStruct((N, D_HID), jnp.float32),
            jax.ShapeDtypeStruct((N, 128), jnp.float32),
        ],
    )(h, stats, g2d, beta2d, wcat, bcat)


def _final(sp, aggp, degp):
    def body(sp_ref, aggp_ref, degp_ref, o_ref):
        deg = jnp.sum(degp_ref[...], axis=0)[:, None]
        inv = 1.0 / jnp.maximum(deg, 1.0)
        p = jnp.sum(aggp_ref[...], axis=0)[:, None]
        o_ref[...] = sp_ref[:, 0:1] + p * inv

    return pl.pallas_call(
        body,
        grid=(NPAD // R,),
        in_specs=[
            pl.BlockSpec((R, 128), lambda i: (i, 0)),
            pl.BlockSpec((N_WORKERS, R), lambda i: (0, i)),
            pl.BlockSpec((N_WORKERS, R), lambda i: (0, i)),
        ],
        out_specs=pl.BlockSpec((R, 1), lambda i: (i, 0)),
        out_shape=jax.ShapeDtypeStruct((N, 1), jnp.float32),
    )(sp, aggp, degp)


def kernel(x, W_self1, W_neigh1, b1, g1, beta1, W_self2, W_neigh2, b2, g2,
           beta2, W_self3, W_neigh3, b3, edge_index):
    src = edge_index[0].astype(jnp.int32)
    dst = edge_index[1].astype(jnp.int32)
    npad_e = EPAD - E
    pad_ar = jnp.arange(npad_e, dtype=jnp.int32)
    src_p = jnp.concatenate([src, pad_ar % N])
    dst_p = jnp.concatenate([dst, N + pad_ar % (NPAD - N)]).reshape(
        EPAD // BLK_E, BLK_E)

    zeros128 = jnp.zeros((NPAD, 128), jnp.float32)
    zeros1d = jnp.zeros((NPAD,), jnp.float32)

    degp = _sc_degree(dst_p, zeros1d)

    def seg128(tbl):
        nc = tbl.shape[1] // 128
        return [
            _make_sc_segsum(128, True, tbl.shape[1], c * 128)(
                tbl, src_p, dst_p, zeros128).reshape(2, NPAD, 128)
            for c in range(nc)
        ]

    def neigh_chain(hself, parts, Wn):
        h = hself
        nc = len(parts)
        for c, pt in enumerate(parts):
            res = _mm_acc(h, pt, degp, Wn[c * 128:(c + 1) * 128], c == nc - 1)
            h = res[0]
        return h, res[1]

    hs1 = _mm_self(x, W_self1, b1.reshape(1, -1))
    parts1 = seg128(x)
    h1, st1 = neigh_chain(hs1, parts1, W_neigh1)
    h1 = _bn_relu(h1, st1, g1.reshape(1, -1), beta1.reshape(1, -1))

    hs2 = _mm_self(h1, W_self2, b2.reshape(1, -1))
    parts2 = seg128(h1)
    h2, st2 = neigh_chain(hs2, parts2, W_neigh2)

    wcat = jnp.zeros((D_HID, 128), jnp.float32)
    wcat = wcat.at[:, 0:1].set(W_self3).at[:, 1:2].set(W_neigh3)
    bcat = jnp.zeros((1, 128), jnp.float32).at[0, 0].set(b3[0])
    h2, sp = _bn_relu_proj(h2, st2, g2.reshape(1, -1), beta2.reshape(1, -1),
                           wcat, bcat)
    p2d = jnp.pad(sp[:, 1], (0, NPAD - N)).reshape(NPAD // 128, 128)
    aggp = _sc_psum(p2d, src_p, dst_p, zeros1d)
    return _final(sp, aggp, degp)


# speedup vs baseline: 1.0690x; 1.0690x over previous
import functools

import jax
import jax.numpy as jnp
from jax import lax
from jax.experimental import pallas as pl
from jax.experimental.pallas import tpu as pltpu
from jax.experimental.pallas import tpu_sc as plsc

N = 10000
E = 160000
D_IN = 256
D_HID = 512
EPS = 1e-5

NPAD = 10240
EPAD = 163840
N_WORKERS = 32
EDGES_PER_W = EPAD // N_WORKERS
BLK_E = 64
N_BLKS = EDGES_PER_W // BLK_E
ROWS_PER_TILE = NPAD // 16
R = 1024


NBUF = 4
N_GRP = N_BLKS // NBUF


@functools.lru_cache(maxsize=None)
def _make_sc_segsum(nc, d_table):
    mesh = plsc.VectorSubcoreMesh(core_axis_name="c", subcore_axis_name="s")
    C = 128

    scratch = [
        pltpu.VMEM((EDGES_PER_W,), jnp.int32),
        pltpu.VMEM((N_BLKS, BLK_E), jnp.int32),
        pltpu.VMEM((NBUF, BLK_E, C), jnp.float32),
        pltpu.VMEM_SHARED((NPAD, C), jnp.float32),
        pltpu.SemaphoreType.DMA,
    ] + [pltpu.SemaphoreType.DMA] * NBUF       + [pltpu.SemaphoreType.DMA] * NBUF

    def body(*refs):
        (table, src_hbm, dst_hbm, zeros_hbm, *rest) = refs
        outs = rest[:nc]
        (srcb, dstb, rows, acc, zsem, *sems) = rest[nc:]
        gsem = sems[:NBUF]
        ssem = sems[NBUF:]

        cid = lax.axis_index("c")
        sid = lax.axis_index("s")
        wid = sid * 2 + cid
        r0 = pl.multiple_of(sid * ROWS_PER_TILE, 8)
        i0 = pl.multiple_of(wid * N_BLKS, 8)
        o0 = pl.multiple_of(cid * NPAD + r0, 8)

        def zero_own_rows():
            return pltpu.make_async_copy(
                zeros_hbm, acc.at[pl.ds(r0, ROWS_PER_TILE)], zsem)

        zdesc = zero_own_rows()
        zdesc.start()
        pltpu.sync_copy(dst_hbm.at[pl.ds(i0, N_BLKS)], dstb)
        e0 = pl.multiple_of(wid * EDGES_PER_W, 8)
        pltpu.sync_copy(src_hbm.at[pl.ds(e0, EDGES_PER_W)], srcb)
        zdesc.wait()
        plsc.subcore_barrier()

        def s_desc(i, b):
            return pltpu.make_async_copy(rows.at[b], acc.at[dstb.at[i]],
                                         ssem[b])

        for c in range(nc):
            col0 = c * C

            def g_desc(i, b):
                off = pl.multiple_of(i * BLK_E, 8)
                return pltpu.make_async_copy(
                    table.at[srcb.at[pl.ds(off, BLK_E)], pl.ds(col0, C)],
                    rows.at[b], gsem[b])

            for b in range(NBUF):
                g_desc(b, b).start()

            def group(g, carry):
                ib = g * NBUF
                for b in range(NBUF):
                    g_desc(ib + b, b).wait()
                    s_desc(ib + b, b).start(add=True)

                @pl.when(g < N_GRP - 1)
                def _():
                    for b in range(NBUF):
                        s_desc(ib + b, b).wait()
                        g_desc(ib + NBUF + b, b).start()
                return carry

            lax.fori_loop(0, N_GRP, group, 0)
            for b in range(NBUF):
                s_desc((N_GRP - 1) * NBUF + b, b).wait()
            plsc.subcore_barrier()

            pltpu.sync_copy(acc.at[pl.ds(r0, ROWS_PER_TILE)],
                            outs[c].at[pl.ds(o0, ROWS_PER_TILE)])
            if c < nc - 1:
                zd = zero_own_rows()
                zd.start()
                zd.wait()
                plsc.subcore_barrier()

    return pl.kernel(
        body,
        out_type=[jax.ShapeDtypeStruct((2 * NPAD, 128), jnp.float32)] * nc,
        mesh=mesh,
        scratch_types=scratch,
    )


def _make_sc_hist(gather):
    mesh = plsc.VectorSubcoreMesh(core_axis_name="c", subcore_axis_name="s")

    scratch = [
        pltpu.VMEM((N_BLKS, BLK_E), jnp.int32),
        pltpu.VMEM((NPAD,), jnp.float32),
    ]
    if gather:
        scratch.insert(0, pltpu.VMEM((EDGES_PER_W,), jnp.int32))
        scratch.append(pltpu.VMEM((NPAD // 128, 128), jnp.float32))

    def body(*refs):
        if gather:
            (ptab_hbm, src_hbm, dst_hbm, zeros_hbm, out_hbm,
             srcb, dstb, acc, ptab) = refs
        else:
            (dst_hbm, zeros_hbm, out_hbm, dstb, acc) = refs

        cid = lax.axis_index("c")
        sid = lax.axis_index("s")
        wid = sid * 2 + cid
        i0 = pl.multiple_of(wid * N_BLKS, 8)

        pltpu.sync_copy(zeros_hbm, acc)
        pltpu.sync_copy(dst_hbm.at[pl.ds(i0, N_BLKS)], dstb)
        if gather:
            e0 = pl.multiple_of(wid * EDGES_PER_W, 8)
            pltpu.sync_copy(src_hbm.at[pl.ds(e0, EDGES_PER_W)], srcb)
            pltpu.sync_copy(ptab_hbm, ptab)
        ones = jnp.full((16,), 1.0, jnp.float32)

        def step(j, carry):
            for l in range(BLK_E // 16):
                dv = dstb[j, pl.ds(l * 16, 16)]
                if gather:
                    sv = srcb[pl.ds(j * BLK_E + l * 16, 16)]
                    pv = plsc.load_gather(
                        ptab, [lax.shift_right_logical(sv, 7),
                               lax.bitwise_and(sv, 127)])
                else:
                    pv = ones
                plsc.addupdate_scatter(acc, [dv], pv)
            return carry

        lax.fori_loop(0, N_BLKS, step, 0)
        pltpu.sync_copy(acc, out_hbm.at[wid])

    return pl.kernel(
        body,
        out_type=jax.ShapeDtypeStruct((N_WORKERS, NPAD), jnp.float32),
        mesh=mesh,
        scratch_types=scratch,
        compiler_params=pltpu.CompilerParams(needs_layout_passes=False),
    )


_sc_degree = _make_sc_hist(gather=False)
_sc_psum = _make_sc_hist(gather=True)


def _mm_self(xin, W, b2d):
    d_in = xin.shape[1]

    def body(x_ref, w_ref, b_ref, o_ref):
        o_ref[...] = jnp.dot(x_ref[...], w_ref[...],
                             preferred_element_type=jnp.float32) + b_ref[...]

    return pl.pallas_call(
        body,
        grid=(NPAD // R,),
        in_specs=[
            pl.BlockSpec((R, d_in), lambda i: (i, 0)),
            pl.BlockSpec((d_in, D_HID), lambda i: (0, 0)),
            pl.BlockSpec((1, D_HID), lambda i: (0, 0)),
        ],
        out_specs=pl.BlockSpec((R, D_HID), lambda i: (i, 0)),
        out_shape=jax.ShapeDtypeStruct((N, D_HID), jnp.float32),
    )(xin, W, b2d)


def _mm_stats(hself, parts, degp, Wn):
    nch = len(parts)

    def body(*refs):
        hs_ref = refs[0]
        part_refs = refs[1:1 + nch]
        degp_ref, wn_ref, h_ref, stats_ref = refs[1 + nch:]
        i = pl.program_id(0)

        agg = jnp.concatenate([p[0] + p[1] for p in part_refs], axis=1)
        deg = jnp.sum(degp_ref[...], axis=0)[:, None]
        inv = 1.0 / jnp.maximum(deg, 1.0)
        h = hs_ref[...] + jnp.dot(agg * inv, wn_ref[...],
                                  preferred_element_type=jnp.float32)
        h_ref[...] = h

        @pl.when(i == 0)
        def _():
            stats_ref[...] = jnp.zeros_like(stats_ref)

        row = i * R + jax.lax.broadcasted_iota(jnp.int32, (R, 1), 0)
        hm = jnp.where(row < N, h, 0.0)
        s1 = jnp.broadcast_to(jnp.sum(hm, axis=0, keepdims=True), (8, D_HID))
        s2 = jnp.broadcast_to(jnp.sum(hm * hm, axis=0, keepdims=True), (8, D_HID))
        stats_ref[...] += jnp.concatenate([s1, s2], axis=0)

    return pl.pallas_call(
        body,
        grid=(NPAD // R,),
        in_specs=[
            pl.BlockSpec((R, D_HID), lambda i: (i, 0)),
            *([pl.BlockSpec((2, R, 128), lambda i: (0, i, 0))] * nch),
            pl.BlockSpec((N_WORKERS, R), lambda i: (0, i)),
            pl.BlockSpec(Wn.shape, lambda i: (0, 0)),
        ],
        out_specs=[
            pl.BlockSpec((R, D_HID), lambda i: (i, 0)),
            pl.BlockSpec((16, D_HID), lambda i: (0, 0)),
        ],
        out_shape=[
            jax.ShapeDtypeStruct((N, D_HID), jnp.float32),
            jax.ShapeDtypeStruct((16, D_HID), jnp.float32),
        ],
    )(hself, *parts, degp, Wn)


def _bn_relu(h, stats, g2d, beta2d):
    def body(h_ref, stats_ref, g_ref, beta_ref, o_ref):
        mu = stats_ref[0:1] * (1.0 / N)
        msq = stats_ref[8:9] * (1.0 / N)
        var = msq - mu * mu
        rstd = 1.0 / jnp.sqrt(var + EPS)
        o_ref[...] = jnp.maximum(
            (h_ref[...] - mu) * (rstd * g_ref[...]) + beta_ref[...], 0.0)

    return pl.pallas_call(
        body,
        grid=(NPAD // R,),
        in_specs=[
            pl.BlockSpec((R, D_HID), lambda i: (i, 0)),
            pl.BlockSpec((16, D_HID), lambda i: (0, 0)),
            pl.BlockSpec((1, D_HID), lambda i: (0, 0)),
            pl.BlockSpec((1, D_HID), lambda i: (0, 0)),
        ],
        out_specs=pl.BlockSpec((R, D_HID), lambda i: (i, 0)),
        out_shape=jax.ShapeDtypeStruct((N, D_HID), jnp.float32),
    )(h, stats, g2d, beta2d)


def _bn_relu_proj(h, stats, g2d, beta2d, wcat, bcat):
    def body(h_ref, stats_ref, g_ref, beta_ref, w_ref, b_ref, o_ref, sp_ref):
        mu = stats_ref[0:1] * (1.0 / N)
        msq = stats_ref[8:9] * (1.0 / N)
        var = msq - mu * mu
        rstd = 1.0 / jnp.sqrt(var + EPS)
        hn = jnp.maximum(
            (h_ref[...] - mu) * (rstd * g_ref[...]) + beta_ref[...], 0.0)
        o_ref[...] = hn
        sp_ref[...] = jnp.dot(hn, w_ref[...],
                              preferred_element_type=jnp.float32) + b_ref[...]

    return pl.pallas_call(
        body,
        grid=(NPAD // R,),
        in_specs=[
            pl.BlockSpec((R, D_HID), lambda i: (i, 0)),
            pl.BlockSpec((16, D_HID), lambda i: (0, 0)),
            pl.BlockSpec((1, D_HID), lambda i: (0, 0)),
            pl.BlockSpec((1, D_HID), lambda i: (0, 0)),
            pl.BlockSpec((D_HID, 128), lambda i: (0, 0)),
            pl.BlockSpec((1, 128), lambda i: (0, 0)),
        ],
        out_specs=[
            pl.BlockSpec((R, D_HID), lambda i: (i, 0)),
            pl.BlockSpec((R, 128), lambda i: (i, 0)),
        ],
        out_shape=[
            jax.ShapeDtypeStruct((N, D_HID), jnp.float32),
            jax.ShapeDtypeStruct((N, 128), jnp.float32),
        ],
    )(h, stats, g2d, beta2d, wcat, bcat)


def _final(sp, aggp, degp):
    def body(sp_ref, aggp_ref, degp_ref, o_ref):
        deg = jnp.sum(degp_ref[...], axis=0)[:, None]
        inv = 1.0 / jnp.maximum(deg, 1.0)
        p = jnp.sum(aggp_ref[...], axis=0)[:, None]
        o_ref[...] = sp_ref[:, 0:1] + p * inv

    return pl.pallas_call(
        body,
        grid=(NPAD // R,),
        in_specs=[
            pl.BlockSpec((R, 128), lambda i: (i, 0)),
            pl.BlockSpec((N_WORKERS, R), lambda i: (0, i)),
            pl.BlockSpec((N_WORKERS, R), lambda i: (0, i)),
        ],
        out_specs=pl.BlockSpec((R, 1), lambda i: (i, 0)),
        out_shape=jax.ShapeDtypeStruct((N, 1), jnp.float32),
    )(sp, aggp, degp)


def kernel(x, W_self1, W_neigh1, b1, g1, beta1, W_self2, W_neigh2, b2, g2,
           beta2, W_self3, W_neigh3, b3, edge_index):
    src = edge_index[0].astype(jnp.int32)
    dst = edge_index[1].astype(jnp.int32)
    npad_e = EPAD - E
    pad_ar = jnp.arange(npad_e, dtype=jnp.int32)
    src_p = jnp.concatenate([src, pad_ar % N])
    dst_p = jnp.concatenate([dst, N + pad_ar % (NPAD - N)]).reshape(
        EPAD // BLK_E, BLK_E)

    zeros128 = jnp.zeros((ROWS_PER_TILE, 128), jnp.float32)
    zeros1d = jnp.zeros((NPAD,), jnp.float32)

    degp = _sc_degree(dst_p, zeros1d)

    def seg128(tbl):
        nc = tbl.shape[1] // 128
        outs = _make_sc_segsum(nc, tbl.shape[1])(tbl, src_p, dst_p, zeros128)
        if nc == 1:
            outs = (outs,)
        return [o.reshape(2, NPAD, 128) for o in outs]

    hs1 = _mm_self(x, W_self1, b1.reshape(1, -1))
    parts1 = seg128(x)
    h1, st1 = _mm_stats(hs1, parts1, degp, W_neigh1)
    h1 = _bn_relu(h1, st1, g1.reshape(1, -1), beta1.reshape(1, -1))

    hs2 = _mm_self(h1, W_self2, b2.reshape(1, -1))
    parts2 = seg128(h1)
    h2, st2 = _mm_stats(hs2, parts2, degp, W_neigh2)

    wcat = jnp.zeros((D_HID, 128), jnp.float32)
    wcat = wcat.at[:, 0:1].set(W_self3).at[:, 1:2].set(W_neigh3)
    bcat = jnp.zeros((1, 128), jnp.float32).at[0, 0].set(b3[0])
    h2, sp = _bn_relu_proj(h2, st2, g2.reshape(1, -1), beta2.reshape(1, -1),
                           wcat, bcat)
    p2d = jnp.pad(sp[:, 1], (0, NPAD - N)).reshape(NPAD // 128, 128)
    aggp = _sc_psum(p2d, src_p, dst_p, zeros1d)
    return _final(sp, aggp, degp)
